# baseline (speedup 1.0000x reference)
import dataclasses

import jax
import jax.numpy as jnp
from jax import lax
from jax.experimental import pallas as pl
from jax.experimental.pallas import tpu as pltpu
from jax.experimental.pallas import tpu_sc as plsc

M = 1536
H = 224
W = 224
N = H * W
NW = 32
ROWS_PER_W = M // NW
HCHUNK = 16
NCHUNK = H // HCHUNK
CODE_CHUNK = HCHUNK * W // 2
VEC = 16
NBUF = 4


def _gather_chunk(row_v, code_v, koff, out_v, slot):
    @pl.loop(0, HCHUNK, step=4)
    def _h(h):

        @plsc.parallel_loop(0, W // 2, step=VEC, unroll=7)
        def _vec(j):
            for hh in range(4):
                w32 = code_v[pl.ds(koff + (h + hh) * (W // 2) + j, VEC)]
                lo = jnp.bitwise_and(w32, 0xFFFF)
                hi = lax.shift_right_logical(w32, 16)
                out_v[slot, h + hh, pl.ds(2 * j, VEC)] = plsc.load_gather(
                    row_v, [lax.shift_right_logical(lo, 8),
                            jnp.bitwise_and(lo, 255)])
                out_v[slot, h + hh, pl.ds(2 * j + VEC, VEC)] = (
                    plsc.load_gather(
                        row_v, [lax.shift_right_logical(hi, 8),
                                jnp.bitwise_and(hi, 255)]))


def _jumble_body(x_hbm, code_hbm, out_hbm, code_v, row_v, out_v,
                 out_sem, fill_sem):
    wid = lax.axis_index("s") * 2 + lax.axis_index("c")
    base = wid * ROWS_PER_W
    pltpu.sync_copy(code_hbm, code_v)

    def out_cp(sl, row, k):
        return pltpu.make_async_copy(
            out_v.at[sl],
            out_hbm.at[row, pl.ds(k * HCHUNK, HCHUNK), :],
            out_sem.at[sl])

    @pl.loop(0, ROWS_PER_W)
    def _rows(r):
        row = base + r
        pltpu.make_async_copy(x_hbm.at[row], row_v, fill_sem).start()
        pltpu.make_async_copy(x_hbm.at[row], row_v, fill_sem).wait()

        for k in range(NCHUNK):
            i = k % NBUF
            if k >= NBUF:
                out_cp(i, row, k - NBUF).wait()
            _gather_chunk(row_v, code_v, k * CODE_CHUNK, out_v, i)
            out_cp(i, row, k).start()
        for k in range(NCHUNK - NBUF, NCHUNK):
            out_cp(k % NBUF, row, k).wait()


def kernel(x, idx):
    b, c, h, w = x.shape
    x3 = x.reshape(M, H, W)
    idx32 = idx.astype(jnp.int32)
    codes = ((idx32 // W) << 8) | (idx32 % W)
    cr = codes.reshape(-1, 2, VEC)
    codeu = cr[:, 0, :] | (cr[:, 1, :] << 16)
    codeu = codeu.reshape(-1)
    mesh = plsc.VectorSubcoreMesh(core_axis_name="c", subcore_axis_name="s")
    cp = pltpu.CompilerParams()
    if "needs_layout_passes" in pltpu.CompilerParams.__dataclass_fields__:
        cp = dataclasses.replace(cp, needs_layout_passes=False)
    run = pl.kernel(
        _jumble_body,
        out_type=jax.ShapeDtypeStruct((M, H, W), jnp.float32),
        mesh=mesh,
        scratch_types=[
            pltpu.VMEM((N // 2,), jnp.int32),
            pltpu.VMEM((H, W), jnp.float32),
            pltpu.VMEM((NBUF, HCHUNK, W), jnp.float32),
            pltpu.SemaphoreType.DMA((NBUF,)),
            pltpu.SemaphoreType.DMA,
        ],
        compiler_params=cp,
    )
    return run(x3, codeu).reshape(b, c, h, w)


# speedup vs baseline: 1.0217x; 1.0217x over previous
import dataclasses

import jax
import jax.numpy as jnp
from jax import lax
from jax.experimental import pallas as pl
from jax.experimental.pallas import tpu as pltpu
from jax.experimental.pallas import tpu_sc as plsc

M = 1536
H = 224
W = 224
N = H * W
NW = 32
ROWS_PER_W = M // NW
HCHUNK = 16
NCHUNK = H // HCHUNK
CODE_CHUNK = HCHUNK * W // 2
VEC = 16
NBUF = 4


def _gather_chunk(row_v, code_v, koff, out_v, slot):
    @pl.loop(0, HCHUNK, step=2)
    def _h(h):

        @plsc.parallel_loop(0, W // 2, step=VEC, unroll=7)
        def _vec(j):
            for hh in range(2):
                w32 = code_v[pl.ds(koff + (h + hh) * (W // 2) + j, VEC)]
                lo = jnp.bitwise_and(w32, 0xFFFF)
                hi = lax.shift_right_logical(w32, 16)
                out_v[slot, h + hh, pl.ds(2 * j, VEC)] = plsc.load_gather(
                    row_v, [lax.shift_right_logical(lo, 8),
                            jnp.bitwise_and(lo, 255)])
                out_v[slot, h + hh, pl.ds(2 * j + VEC, VEC)] = (
                    plsc.load_gather(
                        row_v, [lax.shift_right_logical(hi, 8),
                                jnp.bitwise_and(hi, 255)]))


def _jumble_body(x_hbm, code_hbm, out_hbm, code_v, row_v, out_v,
                 out_sem, fill_sem):
    wid = lax.axis_index("s") * 2 + lax.axis_index("c")
    base = wid * ROWS_PER_W
    pltpu.sync_copy(code_hbm, code_v)

    def out_cp(sl, row, k):
        return pltpu.make_async_copy(
            out_v.at[sl],
            out_hbm.at[row, pl.ds(k * HCHUNK, HCHUNK), :],
            out_sem.at[sl])

    @pl.loop(0, ROWS_PER_W)
    def _rows(r):
        row = base + r
        pltpu.make_async_copy(x_hbm.at[row], row_v, fill_sem).start()
        pltpu.make_async_copy(x_hbm.at[row], row_v, fill_sem).wait()

        for k in range(NCHUNK):
            i = k % NBUF
            if k >= NBUF:
                out_cp(i, row, k - NBUF).wait()
            _gather_chunk(row_v, code_v, k * CODE_CHUNK, out_v, i)
            out_cp(i, row, k).start()
        for k in range(NCHUNK - NBUF, NCHUNK):
            out_cp(k % NBUF, row, k).wait()


def kernel(x, idx):
    b, c, h, w = x.shape
    x3 = x.reshape(M, H, W)
    idx32 = idx.astype(jnp.int32)
    codes = ((idx32 // W) << 8) | (idx32 % W)
    cr = codes.reshape(-1, 2, VEC)
    codeu = cr[:, 0, :] | (cr[:, 1, :] << 16)
    codeu = codeu.reshape(-1)
    mesh = plsc.VectorSubcoreMesh(core_axis_name="c", subcore_axis_name="s")
    cp = pltpu.CompilerParams()
    if "needs_layout_passes" in pltpu.CompilerParams.__dataclass_fields__:
        cp = dataclasses.replace(cp, needs_layout_passes=False)
    run = pl.kernel(
        _jumble_body,
        out_type=jax.ShapeDtypeStruct((M, H, W), jnp.float32),
        mesh=mesh,
        scratch_types=[
            pltpu.VMEM((N // 2,), jnp.int32),
            pltpu.VMEM((H, W), jnp.float32),
            pltpu.VMEM((NBUF, HCHUNK, W), jnp.float32),
            pltpu.SemaphoreType.DMA((NBUF,)),
            pltpu.SemaphoreType.DMA,
        ],
        compiler_params=cp,
    )
    return run(x3, codeu).reshape(b, c, h, w)


# speedup vs baseline: 1.1946x; 1.1693x over previous
import dataclasses

import jax
import jax.numpy as jnp
from jax import lax
from jax.experimental import pallas as pl
from jax.experimental.pallas import tpu as pltpu
from jax.experimental.pallas import tpu_sc as plsc

M = 1536
H = 224
W = 224
N = H * W
NW = 32
ROWS_PER_W = M // NW
HCHUNK = 8
NCHUNK = H // HCHUNK
CODE_CHUNK = HCHUNK * W // 2
VEC = 16
NBUF = 4
NRES = 4


def _gather_chunk(row_v, code_v, koff, out_v, slot):
    @pl.loop(0, HCHUNK)
    def _h(h):

        @plsc.parallel_loop(0, W // 2, step=VEC, unroll=7)
        def _vec(j):
            w32 = code_v[pl.ds(koff + h * (W // 2) + j, VEC)]
            lo = jnp.bitwise_and(w32, 0xFFFF)
            hi = lax.shift_right_logical(w32, 16)
            out_v[slot, h, pl.ds(2 * j, VEC)] = plsc.load_gather(
                row_v, [lax.shift_right_logical(lo, 8),
                        jnp.bitwise_and(lo, 255)])
            out_v[slot, h, pl.ds(2 * j + VEC, VEC)] = plsc.load_gather(
                row_v, [lax.shift_right_logical(hi, 8),
                        jnp.bitwise_and(hi, 255)])


def _jumble_body(x_hbm, code_hbm, out_hbm, code_res, code_ring, row_a,
                 row_b, out_v, out_sem, fill_sem, code_sem):
    wid = lax.axis_index("s") * 2 + lax.axis_index("c")
    base = wid * ROWS_PER_W
    slabs = (row_a, row_b)
    pltpu.sync_copy(code_hbm.at[pl.ds(0, NRES * CODE_CHUNK)], code_res)

    def fill(sl, row):
        return pltpu.make_async_copy(
            x_hbm.at[jnp.minimum(row, M - 1)], slabs[sl], fill_sem.at[sl])

    def ring_cp(rs, k):
        return pltpu.make_async_copy(
            code_hbm.at[pl.ds(k * CODE_CHUNK, CODE_CHUNK)],
            code_ring.at[pl.ds(rs * CODE_CHUNK, CODE_CHUNK)],
            code_sem.at[rs])

    def out_cp(sl, row, k):
        return pltpu.make_async_copy(
            out_v.at[sl],
            out_hbm.at[row, pl.ds(k * HCHUNK, HCHUNK), :],
            out_sem.at[sl])

    fill(0, base).start()

    @pl.loop(0, ROWS_PER_W // 2)
    def _rows(p):
        r0 = base + 2 * p

        for sl, row, nrow in ((0, r0, r0 + 1), (1, r0 + 1, r0 + 2)):
            for u in range(NBUF):
                ring_cp(u, NRES + u).start()
            fill(sl, row).wait()
            fill(1 - sl, nrow).start()

            for k in range(NRES):
                i = k % NBUF
                _gather_chunk(slabs[sl], code_res, k * CODE_CHUNK,
                              out_v, i)
                out_cp(i, row, k).start()

            @pl.loop(0, (NCHUNK - NRES) // NBUF)
            def _stream(g, sl=sl, row=row):
                for u in range(NBUF):
                    k = NRES + g * NBUF + u
                    ring_cp(u, k).wait()
                    out_cp(u, row, k - NBUF).wait()
                    _gather_chunk(slabs[sl], code_ring, u * CODE_CHUNK,
                                  out_v, u)
                    out_cp(u, row, k).start()

                    @pl.when(k + NBUF < NCHUNK)
                    def _():
                        ring_cp(u, k + NBUF).start()

            for k in range(NCHUNK - NBUF, NCHUNK):
                out_cp(k % NBUF, row, k).wait()


def kernel(x, idx):
    b, c, h, w = x.shape
    x3 = x.reshape(M, H, W)
    idx32 = idx.astype(jnp.int32)
    codes = ((idx32 // W) << 8) | (idx32 % W)
    cr = codes.reshape(-1, 2, VEC)
    codeu = cr[:, 0, :] | (cr[:, 1, :] << 16)
    codeu = codeu.reshape(-1)
    mesh = plsc.VectorSubcoreMesh(core_axis_name="c", subcore_axis_name="s")
    cp = pltpu.CompilerParams()
    if "needs_layout_passes" in pltpu.CompilerParams.__dataclass_fields__:
        cp = dataclasses.replace(cp, needs_layout_passes=False)
    run = pl.kernel(
        _jumble_body,
        out_type=jax.ShapeDtypeStruct((M, H, W), jnp.float32),
        mesh=mesh,
        scratch_types=[
            pltpu.VMEM((NRES * CODE_CHUNK,), jnp.int32),
            pltpu.VMEM((NBUF * CODE_CHUNK,), jnp.int32),
            pltpu.VMEM((H, W), jnp.float32),
            pltpu.VMEM((H, W), jnp.float32),
            pltpu.VMEM((NBUF, HCHUNK, W), jnp.float32),
            pltpu.SemaphoreType.DMA((NBUF,)),
            pltpu.SemaphoreType.DMA((2,)),
            pltpu.SemaphoreType.DMA((NBUF,)),
        ],
        compiler_params=cp,
    )
    return run(x3, codeu).reshape(b, c, h, w)
